# baseline (device time: 25746 ns/iter reference)
import jax
import jax.numpy as jnp
from jax import lax
from jax.experimental import pallas as pl
from jax.experimental.pallas import tpu as pltpu

N_DEV = 16
B, SQ, SKV, HQ, DH = 2, 256, 256, 64, 64
H_LOC = HQ // N_DEV
D_HEADS = H_LOC * DH
D_MODEL = 512
CHUNK = SQ // N_DEV
NEG = -1e9


def _block_mask():
    qb = lax.broadcasted_iota(jnp.int32, (SQ, SKV), 0) // 64
    kb = lax.broadcasted_iota(jnp.int32, (SQ, SKV), 1) // 64
    return (qb == kb) | (kb == 0) | ((qb + kb) % 3 == 0)


def kernel(x, Wq, K_ext, V_ext, Wo):
    idx = lax.axis_index("i")
    k_loc = lax.dynamic_slice_in_dim(K_ext, idx * H_LOC, H_LOC, axis=2)
    v_loc = lax.dynamic_slice_in_dim(V_ext, idx * H_LOC, H_LOC, axis=2)
    k_loc = k_loc.astype(jnp.bfloat16).reshape(B, SKV, D_HEADS)
    v_loc = v_loc.astype(jnp.bfloat16).reshape(B, SKV, D_HEADS)

    def body(x_hbm, wq_hbm, k_hbm, v_hbm, wo_hbm, out_ref,
             x_s, wq_s, k_s, v_s, wo_s, part_ref, red_ref, rs_buf,
             rs_send, rs_recv, ag_send, ag_recv, in_sem, st_sem):
        my = lax.axis_index("i")

        loads = [
            pltpu.make_async_copy(x_hbm, x_s, in_sem.at[0]),
            pltpu.make_async_copy(wq_hbm, wq_s, in_sem.at[1]),
            pltpu.make_async_copy(k_hbm, k_s, in_sem.at[2]),
            pltpu.make_async_copy(v_hbm, v_s, in_sem.at[3]),
            pltpu.make_async_copy(wo_hbm, wo_s, in_sem.at[4]),
        ]
        for cp in loads:
            cp.start()

        barrier = pltpu.get_barrier_semaphore()
        for d in range(1, N_DEV):
            pl.semaphore_signal(barrier, inc=1,
                                device_id=((my + d) % N_DEV,),
                                device_id_type=pl.DeviceIdType.MESH)
        for cp in loads:
            cp.wait()

        mask = _block_mask()
        wq = (wq_s[...] * 0.125).astype(jnp.bfloat16)
        wo = wo_s[...].astype(jnp.bfloat16)
        for b in range(B):
            q = jnp.dot(x_s[b].astype(jnp.bfloat16), wq,
                        preferred_element_type=jnp.float32)
            q = q.astype(jnp.bfloat16)
            ctx = []
            for h in range(H_LOC):
                qh = q[:, h * DH:(h + 1) * DH]
                s = lax.dot_general(
                    qh, k_s[b][:, h * DH:(h + 1) * DH],
                    (((1,), (1,)), ((), ())),
                    preferred_element_type=jnp.float32)
                w = jnp.exp(jnp.where(mask, s, NEG))
                w = w / jnp.sum(w, axis=1, keepdims=True)
                ctx.append(jnp.dot(w.astype(jnp.bfloat16),
                                   v_s[b][:, h * DH:(h + 1) * DH],
                                   preferred_element_type=jnp.float32))
            ctx = jnp.concatenate(ctx, axis=1).astype(jnp.bfloat16)
            part_ref[b] = jnp.dot(ctx, wo,
                                  preferred_element_type=jnp.float32
                                  ).astype(jnp.bfloat16)

            if b == 0:
                pl.semaphore_wait(barrier, N_DEV - 1)
            for d in range(1, N_DEV):
                t = (my + d) % N_DEV
                r = N_DEV - d
                pltpu.make_async_remote_copy(
                    src_ref=part_ref.at[b, pl.ds(t * CHUNK, CHUNK), :],
                    dst_ref=rs_buf.at[r, b],
                    send_sem=rs_send.at[r, b],
                    recv_sem=rs_recv.at[r, b],
                    device_id=(t,),
                    device_id_type=pl.DeviceIdType.MESH,
                ).start()

        for b in range(B):
            for r in range(1, N_DEV):
                pltpu.make_async_remote_copy(
                    src_ref=rs_buf.at[r, b], dst_ref=rs_buf.at[r, b],
                    send_sem=rs_send.at[r, b], recv_sem=rs_recv.at[r, b],
                    device_id=((my + r) % N_DEV,),
                    device_id_type=pl.DeviceIdType.MESH,
                ).wait_recv()

            acc = part_ref[b, pl.ds(my * CHUNK, CHUNK), :].astype(jnp.float32)
            for r in range(1, N_DEV):
                acc = acc + rs_buf[r, b].astype(jnp.float32)
            red_ref[b] = acc.astype(jnp.bfloat16)

            pltpu.make_async_copy(
                red_ref.at[b],
                out_ref.at[b, pl.ds(my * CHUNK, CHUNK), :],
                st_sem.at[b],
            ).start()

            for d in range(1, N_DEV):
                t = (my + d) % N_DEV
                r = N_DEV - d
                pltpu.make_async_remote_copy(
                    src_ref=red_ref.at[b],
                    dst_ref=out_ref.at[b, pl.ds(my * CHUNK, CHUNK), :],
                    send_sem=ag_send.at[r, b],
                    recv_sem=ag_recv.at[r, b],
                    device_id=(t,),
                    device_id_type=pl.DeviceIdType.MESH,
                ).start()

        for b in range(B):
            for r in range(1, N_DEV):
                s_dev = (my + r) % N_DEV
                pltpu.make_async_remote_copy(
                    src_ref=red_ref.at[b],
                    dst_ref=out_ref.at[b, pl.ds(s_dev * CHUNK, CHUNK), :],
                    send_sem=ag_send.at[r, b],
                    recv_sem=ag_recv.at[r, b],
                    device_id=(s_dev,),
                    device_id_type=pl.DeviceIdType.MESH,
                ).wait_recv()

        for b in range(B):
            pltpu.make_async_copy(
                red_ref.at[b],
                out_ref.at[b, pl.ds(my * CHUNK, CHUNK), :],
                st_sem.at[b],
            ).wait()
            for r in range(1, N_DEV):
                pltpu.make_async_remote_copy(
                    src_ref=part_ref.at[b, pl.ds(0, CHUNK), :],
                    dst_ref=rs_buf.at[r, b],
                    send_sem=rs_send.at[r, b], recv_sem=rs_recv.at[r, b],
                    device_id=((my + r) % N_DEV,),
                    device_id_type=pl.DeviceIdType.MESH,
                ).wait_send()
                pltpu.make_async_remote_copy(
                    src_ref=red_ref.at[b],
                    dst_ref=rs_buf.at[r, b],
                    send_sem=ag_send.at[r, b], recv_sem=ag_recv.at[r, b],
                    device_id=((my + r) % N_DEV,),
                    device_id_type=pl.DeviceIdType.MESH,
                ).wait_send()

    return pl.pallas_call(
        body,
        out_shape=jax.ShapeDtypeStruct((B, SQ, D_MODEL), jnp.bfloat16),
        in_specs=[pl.BlockSpec(memory_space=pl.ANY)] * 5,
        out_specs=pl.BlockSpec(memory_space=pl.ANY),
        scratch_shapes=[
            pltpu.VMEM((B, SQ, D_MODEL), jnp.float32),
            pltpu.VMEM((D_MODEL, D_HEADS), jnp.float32),
            pltpu.VMEM((B, SKV, D_HEADS), jnp.bfloat16),
            pltpu.VMEM((B, SKV, D_HEADS), jnp.bfloat16),
            pltpu.VMEM((D_HEADS, D_MODEL), jnp.float32),
            pltpu.VMEM((B, SQ, D_MODEL), jnp.bfloat16),
            pltpu.VMEM((B, CHUNK, D_MODEL), jnp.bfloat16),
            pltpu.VMEM((N_DEV, B, CHUNK, D_MODEL), jnp.bfloat16),
            pltpu.SemaphoreType.DMA((N_DEV, B)),
            pltpu.SemaphoreType.DMA((N_DEV, B)),
            pltpu.SemaphoreType.DMA((N_DEV, B)),
            pltpu.SemaphoreType.DMA((N_DEV, B)),
            pltpu.SemaphoreType.DMA((5,)),
            pltpu.SemaphoreType.DMA((B,)),
        ],
        compiler_params=pltpu.CompilerParams(collective_id=0),
    )(x, Wq, k_loc, v_loc, Wo)


# device time: 25733 ns/iter; 1.0005x vs baseline; 1.0005x over previous
import jax
import jax.numpy as jnp
from jax import lax
from jax.experimental import pallas as pl
from jax.experimental.pallas import tpu as pltpu

N_DEV = 16
B, SQ, SKV, HQ, DH = 2, 256, 256, 64, 64
H_LOC = HQ // N_DEV
D_HEADS = H_LOC * DH
D_MODEL = 512
CHUNK = SQ // N_DEV
NEG = -1e9


def _block_mask():
    qb = lax.broadcasted_iota(jnp.int32, (SQ, SKV), 0) // 64
    kb = lax.broadcasted_iota(jnp.int32, (SQ, SKV), 1) // 64
    return (qb == kb) | (kb == 0) | ((qb + kb) % 3 == 0)


def kernel(x, Wq, K_ext, V_ext, Wo):
    idx = lax.axis_index("i")
    k_loc = lax.dynamic_slice_in_dim(K_ext, idx * H_LOC, H_LOC, axis=2)
    v_loc = lax.dynamic_slice_in_dim(V_ext, idx * H_LOC, H_LOC, axis=2)
    k_loc = k_loc.astype(jnp.bfloat16).reshape(B, SKV, D_HEADS)
    v_loc = v_loc.astype(jnp.bfloat16).reshape(B, SKV, D_HEADS)
    xb = x.astype(jnp.bfloat16)
    wqb = (Wq * 0.125).astype(jnp.bfloat16)
    wob = Wo.astype(jnp.bfloat16)

    def body(x_ref, wq_ref, k_hbm, v_hbm, wo_ref, out_ref,
             k_s, v_s, part_ref, red_ref, rs_buf,
             rs_send, rs_recv, ag_send, ag_recv, in_sem, st_sem):
        my = lax.axis_index("i")

        loads = [
            pltpu.make_async_copy(k_hbm, k_s, in_sem.at[0]),
            pltpu.make_async_copy(v_hbm, v_s, in_sem.at[1]),
        ]
        for cp in loads:
            cp.start()

        barrier = pltpu.get_barrier_semaphore()
        for d in range(1, N_DEV):
            pl.semaphore_signal(barrier, inc=1,
                                device_id=((my + d) % N_DEV,),
                                device_id_type=pl.DeviceIdType.MESH)
        for cp in loads:
            cp.wait()

        mask = _block_mask()
        wq = wq_ref[...]
        wo = wo_ref[...]
        for b in range(B):
            q = jnp.dot(x_ref[b], wq,
                        preferred_element_type=jnp.float32)
            q = q.astype(jnp.bfloat16)
            ctx = []
            for h in range(H_LOC):
                qh = q[:, h * DH:(h + 1) * DH]
                s = lax.dot_general(
                    qh, k_s[b][:, h * DH:(h + 1) * DH],
                    (((1,), (1,)), ((), ())),
                    preferred_element_type=jnp.float32)
                w = jnp.exp(jnp.where(mask, s, NEG))
                w = w / jnp.sum(w, axis=1, keepdims=True)
                ctx.append(jnp.dot(w.astype(jnp.bfloat16),
                                   v_s[b][:, h * DH:(h + 1) * DH],
                                   preferred_element_type=jnp.float32))
            ctx = jnp.concatenate(ctx, axis=1).astype(jnp.bfloat16)
            part_ref[b] = jnp.dot(ctx, wo,
                                  preferred_element_type=jnp.float32
                                  ).astype(jnp.bfloat16)

            if b == 0:
                pl.semaphore_wait(barrier, N_DEV - 1)
            for d in range(1, N_DEV):
                t = (my + d) % N_DEV
                r = N_DEV - d
                pltpu.make_async_remote_copy(
                    src_ref=part_ref.at[b, pl.ds(t * CHUNK, CHUNK), :],
                    dst_ref=rs_buf.at[r, b],
                    send_sem=rs_send.at[r, b],
                    recv_sem=rs_recv.at[r, b],
                    device_id=(t,),
                    device_id_type=pl.DeviceIdType.MESH,
                ).start()

        for b in range(B):
            for r in range(1, N_DEV):
                pltpu.make_async_remote_copy(
                    src_ref=rs_buf.at[r, b], dst_ref=rs_buf.at[r, b],
                    send_sem=rs_send.at[r, b], recv_sem=rs_recv.at[r, b],
                    device_id=((my + r) % N_DEV,),
                    device_id_type=pl.DeviceIdType.MESH,
                ).wait_recv()

            acc = part_ref[b, pl.ds(my * CHUNK, CHUNK), :].astype(jnp.float32)
            for r in range(1, N_DEV):
                acc = acc + rs_buf[r, b].astype(jnp.float32)
            red_ref[b] = acc.astype(jnp.bfloat16)

            pltpu.make_async_copy(
                red_ref.at[b],
                out_ref.at[b, pl.ds(my * CHUNK, CHUNK), :],
                st_sem.at[b],
            ).start()

            for d in range(1, N_DEV):
                t = (my + d) % N_DEV
                r = N_DEV - d
                pltpu.make_async_remote_copy(
                    src_ref=red_ref.at[b],
                    dst_ref=out_ref.at[b, pl.ds(my * CHUNK, CHUNK), :],
                    send_sem=ag_send.at[r, b],
                    recv_sem=ag_recv.at[r, b],
                    device_id=(t,),
                    device_id_type=pl.DeviceIdType.MESH,
                ).start()

        for b in range(B):
            for r in range(1, N_DEV):
                s_dev = (my + r) % N_DEV
                pltpu.make_async_remote_copy(
                    src_ref=red_ref.at[b],
                    dst_ref=out_ref.at[b, pl.ds(s_dev * CHUNK, CHUNK), :],
                    send_sem=ag_send.at[r, b],
                    recv_sem=ag_recv.at[r, b],
                    device_id=(s_dev,),
                    device_id_type=pl.DeviceIdType.MESH,
                ).wait_recv()

        for b in range(B):
            pltpu.make_async_copy(
                red_ref.at[b],
                out_ref.at[b, pl.ds(my * CHUNK, CHUNK), :],
                st_sem.at[b],
            ).wait()
            for r in range(1, N_DEV):
                pltpu.make_async_remote_copy(
                    src_ref=part_ref.at[b, pl.ds(0, CHUNK), :],
                    dst_ref=rs_buf.at[r, b],
                    send_sem=rs_send.at[r, b], recv_sem=rs_recv.at[r, b],
                    device_id=((my + r) % N_DEV,),
                    device_id_type=pl.DeviceIdType.MESH,
                ).wait_send()
                pltpu.make_async_remote_copy(
                    src_ref=red_ref.at[b],
                    dst_ref=rs_buf.at[r, b],
                    send_sem=ag_send.at[r, b], recv_sem=ag_recv.at[r, b],
                    device_id=((my + r) % N_DEV,),
                    device_id_type=pl.DeviceIdType.MESH,
                ).wait_send()

    return pl.pallas_call(
        body,
        out_shape=jax.ShapeDtypeStruct((B, SQ, D_MODEL), jnp.bfloat16),
        in_specs=[
            pl.BlockSpec(memory_space=pltpu.VMEM),
            pl.BlockSpec(memory_space=pltpu.VMEM),
            pl.BlockSpec(memory_space=pl.ANY),
            pl.BlockSpec(memory_space=pl.ANY),
            pl.BlockSpec(memory_space=pltpu.VMEM),
        ],
        out_specs=pl.BlockSpec(memory_space=pl.ANY),
        scratch_shapes=[
            pltpu.VMEM((B, SKV, D_HEADS), jnp.bfloat16),
            pltpu.VMEM((B, SKV, D_HEADS), jnp.bfloat16),
            pltpu.VMEM((B, SQ, D_MODEL), jnp.bfloat16),
            pltpu.VMEM((B, CHUNK, D_MODEL), jnp.bfloat16),
            pltpu.VMEM((N_DEV, B, CHUNK, D_MODEL), jnp.bfloat16),
            pltpu.SemaphoreType.DMA((N_DEV, B)),
            pltpu.SemaphoreType.DMA((N_DEV, B)),
            pltpu.SemaphoreType.DMA((N_DEV, B)),
            pltpu.SemaphoreType.DMA((N_DEV, B)),
            pltpu.SemaphoreType.DMA((2,)),
            pltpu.SemaphoreType.DMA((B,)),
        ],
        compiler_params=pltpu.CompilerParams(collective_id=0),
    )(xb, wqb, k_loc, v_loc, wob)


# device time: 25722 ns/iter; 1.0009x vs baseline; 1.0004x over previous
import jax
import jax.numpy as jnp
from jax import lax
from jax.experimental import pallas as pl
from jax.experimental.pallas import tpu as pltpu

N_DEV = 16
B, SQ, SKV, HQ, DH = 2, 256, 256, 64, 64
H_LOC = HQ // N_DEV
D_HEADS = H_LOC * DH
D_MODEL = 512
CHUNK = SQ // N_DEV
NEG = -1e9


def _block_mask():
    qb = lax.broadcasted_iota(jnp.int32, (SQ, SKV), 0) // 64
    kb = lax.broadcasted_iota(jnp.int32, (SQ, SKV), 1) // 64
    return (qb == kb) | (kb == 0) | ((qb + kb) % 3 == 0)


def kernel(x, Wq, K_ext, V_ext, Wo):
    idx = lax.axis_index("i")
    k_loc = lax.dynamic_slice_in_dim(K_ext, idx * H_LOC, H_LOC, axis=2)
    v_loc = lax.dynamic_slice_in_dim(V_ext, idx * H_LOC, H_LOC, axis=2)
    k_loc = k_loc.astype(jnp.bfloat16).reshape(B, SKV, D_HEADS)
    v_loc = v_loc.astype(jnp.bfloat16).reshape(B, SKV, D_HEADS)

    def body(x_hbm, wq_hbm, k_hbm, v_hbm, wo_hbm, out_ref,
             x_s, wq_s, k_s, v_s, wo_s, part_ref, red_ref, rs_buf,
             rs_send, rs_recv, ag_send, ag_recv, in_sem, st_sem):
        my = lax.axis_index("i")

        loads = [
            pltpu.make_async_copy(x_hbm, x_s, in_sem.at[0]),
            pltpu.make_async_copy(wq_hbm, wq_s, in_sem.at[1]),
            pltpu.make_async_copy(k_hbm, k_s, in_sem.at[2]),
            pltpu.make_async_copy(v_hbm, v_s, in_sem.at[3]),
            pltpu.make_async_copy(wo_hbm, wo_s, in_sem.at[4]),
        ]
        for cp in loads:
            cp.start()

        barrier = pltpu.get_barrier_semaphore()
        for d in range(1, N_DEV):
            pl.semaphore_signal(barrier, inc=1,
                                device_id=((my + d) % N_DEV,),
                                device_id_type=pl.DeviceIdType.MESH)
        for cp in loads:
            cp.wait()

        mask = _block_mask()
        wq = (wq_s[...] * 0.125).astype(jnp.bfloat16)
        wo = wo_s[...].astype(jnp.bfloat16)
        for b in range(B):
            q = jnp.dot(x_s[b].astype(jnp.bfloat16), wq,
                        preferred_element_type=jnp.float32)
            q = q.astype(jnp.bfloat16)
            ctx = []
            for h in range(H_LOC):
                qh = q[:, h * DH:(h + 1) * DH]
                s = lax.dot_general(
                    qh, k_s[b][:, h * DH:(h + 1) * DH],
                    (((1,), (1,)), ((), ())),
                    preferred_element_type=jnp.float32)
                w = jnp.exp(jnp.where(mask, s, NEG))
                w = w / jnp.sum(w, axis=1, keepdims=True)
                ctx.append(jnp.dot(w.astype(jnp.bfloat16),
                                   v_s[b][:, h * DH:(h + 1) * DH],
                                   preferred_element_type=jnp.float32))
            ctx = jnp.concatenate(ctx, axis=1).astype(jnp.bfloat16)
            part_ref[b] = jnp.dot(ctx, wo,
                                  preferred_element_type=jnp.float32
                                  ).astype(jnp.bfloat16)

            if b == 0:
                pl.semaphore_wait(barrier, N_DEV - 1)
            for d in range(1, N_DEV):
                t = (my + d) % N_DEV
                r = N_DEV - d
                pltpu.make_async_remote_copy(
                    src_ref=part_ref.at[b, pl.ds(t * CHUNK, CHUNK), :],
                    dst_ref=rs_buf.at[r, b],
                    send_sem=rs_send.at[r, b],
                    recv_sem=rs_recv.at[r, b],
                    device_id=(t,),
                    device_id_type=pl.DeviceIdType.MESH,
                ).start()

        for b in range(B):
            for r in range(1, N_DEV):
                pltpu.make_async_remote_copy(
                    src_ref=rs_buf.at[r, b], dst_ref=rs_buf.at[r, b],
                    send_sem=rs_send.at[r, b], recv_sem=rs_recv.at[r, b],
                    device_id=((my + r) % N_DEV,),
                    device_id_type=pl.DeviceIdType.MESH,
                ).wait_recv()

            acc = part_ref[b, pl.ds(my * CHUNK, CHUNK), :].astype(jnp.float32)
            for r in range(1, N_DEV):
                acc = acc + rs_buf[r, b].astype(jnp.float32)
            red_ref[b] = acc.astype(jnp.bfloat16)

            pltpu.make_async_copy(
                red_ref.at[b],
                out_ref.at[b, pl.ds(my * CHUNK, CHUNK), :],
                st_sem.at[b],
            ).start()

            for d in range(1, N_DEV):
                t = (my + d) % N_DEV
                r = N_DEV - d
                pltpu.make_async_remote_copy(
                    src_ref=red_ref.at[b],
                    dst_ref=out_ref.at[b, pl.ds(my * CHUNK, CHUNK), :],
                    send_sem=ag_send.at[r, b],
                    recv_sem=ag_recv.at[r, b],
                    device_id=(t,),
                    device_id_type=pl.DeviceIdType.MESH,
                ).start()

        for b in range(B):
            for r in range(1, N_DEV):
                s_dev = (my + r) % N_DEV
                pltpu.make_async_remote_copy(
                    src_ref=red_ref.at[b],
                    dst_ref=out_ref.at[b, pl.ds(s_dev * CHUNK, CHUNK), :],
                    send_sem=ag_send.at[r, b],
                    recv_sem=ag_recv.at[r, b],
                    device_id=(s_dev,),
                    device_id_type=pl.DeviceIdType.MESH,
                ).wait_recv()

        for b in range(B):
            pltpu.make_async_copy(
                red_ref.at[b],
                out_ref.at[b, pl.ds(my * CHUNK, CHUNK), :],
                st_sem.at[b],
            ).wait()
            for r in range(1, N_DEV):
                pltpu.make_async_remote_copy(
                    src_ref=part_ref.at[b, pl.ds(0, CHUNK), :],
                    dst_ref=rs_buf.at[r, b],
                    send_sem=rs_send.at[r, b], recv_sem=rs_recv.at[r, b],
                    device_id=((my + r) % N_DEV,),
                    device_id_type=pl.DeviceIdType.MESH,
                ).wait_send()
                pltpu.make_async_remote_copy(
                    src_ref=red_ref.at[b],
                    dst_ref=rs_buf.at[r, b],
                    send_sem=ag_send.at[r, b], recv_sem=ag_recv.at[r, b],
                    device_id=((my + r) % N_DEV,),
                    device_id_type=pl.DeviceIdType.MESH,
                ).wait_send()

    return pl.pallas_call(
        body,
        out_shape=jax.ShapeDtypeStruct((B, SQ, D_MODEL), jnp.bfloat16),
        in_specs=[pl.BlockSpec(memory_space=pl.ANY)] * 5,
        out_specs=pl.BlockSpec(memory_space=pl.ANY),
        scratch_shapes=[
            pltpu.VMEM((B, SQ, D_MODEL), jnp.float32),
            pltpu.VMEM((D_MODEL, D_HEADS), jnp.float32),
            pltpu.VMEM((B, SKV, D_HEADS), jnp.bfloat16),
            pltpu.VMEM((B, SKV, D_HEADS), jnp.bfloat16),
            pltpu.VMEM((D_HEADS, D_MODEL), jnp.float32),
            pltpu.VMEM((B, SQ, D_MODEL), jnp.bfloat16),
            pltpu.VMEM((B, CHUNK, D_MODEL), jnp.bfloat16),
            pltpu.VMEM((N_DEV, B, CHUNK, D_MODEL), jnp.bfloat16),
            pltpu.SemaphoreType.DMA((N_DEV, B)),
            pltpu.SemaphoreType.DMA((N_DEV, B)),
            pltpu.SemaphoreType.DMA((N_DEV, B)),
            pltpu.SemaphoreType.DMA((N_DEV, B)),
            pltpu.SemaphoreType.DMA((5,)),
            pltpu.SemaphoreType.DMA((B,)),
        ],
        compiler_params=pltpu.CompilerParams(collective_id=0),
    )(x, Wq, k_loc, v_loc, Wo)
